# baseline (speedup 1.0000x reference)
import functools

import jax
import jax.numpy as jnp
from jax import lax
from jax.experimental import pallas as pl
from jax.experimental.pallas import tpu as pltpu
from jax.experimental.pallas import tpu_sc as plsc

N = 8192
DIM = 4096
K = 3

BLK = 512
GRID_F = N // BLK
SUB = BLK // 128
NROWS128 = N // 128

LANES = 16
NSUB = 16
NPT = N // NSUB
NCHUNK = NPT // LANES
GPAD = 16


def _norms_body(h_blk, out_ref):
    i = pl.program_id(0)
    x = h_blk[...]
    s = jnp.sum(x * x, axis=1)
    out_ref[pl.ds(i * SUB, SUB), :] = s.reshape(SUB, 128)


def _local_top3(nv, gbase, lane):
    vals, gidxs = [], []
    excl = []

    for _ in range(K):
        def step(c, carry):
            rm, ri = carry
            x = nv[pl.ds(c * LANES, LANES)]
            gi = gbase + c * LANES + lane
            for e in excl:
                x = jnp.where(gi == e, jnp.float32(-1.0), x)
            upd = x > rm
            return jnp.where(upd, x, rm), jnp.where(upd, gi, ri)

        rm, ri = lax.fori_loop(
            0, NCHUNK, step,
            (jnp.full((LANES,), -1.0, jnp.float32),
             jnp.zeros((LANES,), jnp.int32)))
        m = jnp.max(rm)
        big = jnp.int32(2**31 - 1)
        ik = jnp.min(jnp.where((rm == m) & (m > -1.0), ri, big))
        vals.append(m)
        gidxs.append(ik)
        excl = gidxs
    return vals, gidxs


def _sc_select_body(norms_hbm, tokens_hbm, hid_hbm,
                    anchors_hbm, meta_hbm,
                    nv, tokv, vals_sh, idxs_sh,
                    cand_v, cand_i, idx8, rows_v, outmeta, sem):
    sid = lax.axis_index("s")
    cid = lax.axis_index("c")
    lane = lax.iota(jnp.int32, LANES)

    pltpu.sync_copy(norms_hbm.at[pl.ds(sid * NPT, NPT)], nv)
    vals, gidxs = _local_top3(nv, sid * NPT, lane)

    vvec = jnp.where(lane == 0, vals[0],
                     jnp.where(lane == 1, vals[1],
                               jnp.where(lane == 2, vals[2], -1.0)))
    ivec = jnp.where(lane == 0, gidxs[0],
                     jnp.where(lane == 1, gidxs[1],
                               jnp.where(lane == 2, gidxs[2], 0)))
    nv[pl.ds(0, LANES)] = vvec
    pltpu.sync_copy(nv.at[pl.ds(0, LANES)],
                    vals_sh.at[pl.ds(sid * LANES, LANES)])
    nv[pl.ds(0, LANES)] = plsc.bitcast(ivec, jnp.float32)
    pltpu.sync_copy(nv.at[pl.ds(0, LANES)],
                    idxs_sh.at[pl.ds(sid * LANES, LANES)])
    plsc.subcore_barrier()

    @pl.when((sid == 0) & (cid == 0))
    def _():
        pltpu.sync_copy(vals_sh, cand_v)
        pltpu.sync_copy(idxs_sh, cand_i)

        big = jnp.int32(2**31 - 1)
        sel = []
        excl = []
        for _ in range(K):
            def step(c, carry):
                rm, ri = carry
                x = cand_v[pl.ds(c * LANES, LANES)]
                gi = plsc.bitcast(cand_i[pl.ds(c * LANES, LANES)], jnp.int32)
                for e in excl:
                    x = jnp.where(gi == e, jnp.float32(-1.0), x)
                upd = x > rm
                return jnp.where(upd, x, rm), jnp.where(upd, gi, ri)

            rm, ri = lax.fori_loop(
                0, NSUB, step,
                (jnp.full((LANES,), -1.0, jnp.float32),
                 jnp.zeros((LANES,), jnp.int32)))
            m = jnp.max(rm)
            cand_all = jnp.where(rm == m, ri, big)
            def tie_step(c, best):
                x = cand_v[pl.ds(c * LANES, LANES)]
                gi = plsc.bitcast(cand_i[pl.ds(c * LANES, LANES)], jnp.int32)
                for e in excl:
                    x = jnp.where(gi == e, jnp.float32(-1.0), x)
                hit = jnp.where(x == m, gi, big)
                return jnp.minimum(best, jnp.min(hit))
            ik = lax.fori_loop(0, NSUB, tie_step, jnp.min(cand_all))
            sel.append(ik)
            excl = sel

        pltpu.sync_copy(tokens_hbm, tokv)
        tidx = jnp.where(lane == 0, sel[0],
                         jnp.where(lane == 1, sel[1],
                                   jnp.where(lane == 2, sel[2], 0)))
        tsel = plsc.load_gather(tokv, [tidx])
        outmeta[pl.ds(0, LANES)] = tsel
        pltpu.sync_copy(outmeta, meta_hbm)

        gvec = jnp.where(lane == 0, sel[0],
                         jnp.where(lane == 1, sel[1],
                                   jnp.where(lane == 2, sel[2], lane)))
        idx8[pl.ds(0, GPAD)] = gvec
        pltpu.async_copy(hid_hbm.at[idx8], rows_v, sem).wait()
        pltpu.sync_copy(rows_v.at[pl.ds(0, K), :], anchors_hbm)


def _make_sc_select():
    return functools.partial(
        pl.kernel,
        mesh=plsc.VectorSubcoreMesh(core_axis_name="c", subcore_axis_name="s"),
        out_type=[
            jax.ShapeDtypeStruct((K, DIM), jnp.float32),
            jax.ShapeDtypeStruct((LANES,), jnp.int32),
        ],
        scratch_types=[
            pltpu.VMEM((NPT,), jnp.float32),
            pltpu.VMEM((N,), jnp.int32),
            pltpu.VMEM_SHARED((NSUB * LANES,), jnp.float32),
            pltpu.VMEM_SHARED((NSUB * LANES,), jnp.float32),
            pltpu.VMEM((NSUB * LANES,), jnp.float32),
            pltpu.VMEM((NSUB * LANES,), jnp.float32),
            pltpu.VMEM((GPAD,), jnp.int32),
            pltpu.VMEM((GPAD, DIM), jnp.float32),
            pltpu.VMEM((LANES,), jnp.int32),
            pltpu.SemaphoreType.DMA,
        ],
        compiler_params=pltpu.CompilerParams(needs_layout_passes=False),
    )(_sc_select_body)


_SC_SELECT = None


def _sc_select(norms_flat, tokens, hidden):
    global _SC_SELECT
    if _SC_SELECT is None:
        _SC_SELECT = _make_sc_select()
    return _SC_SELECT(norms_flat, tokens, hidden)


@jax.jit
def _run(hidden, tokens_i32):
    norms2 = pl.pallas_call(
        _norms_body,
        grid=(GRID_F,),
        in_specs=[pl.BlockSpec((BLK, DIM), lambda i: (i, 0))],
        out_specs=pl.BlockSpec((NROWS128, 128), lambda i: (0, 0)),
        out_shape=jax.ShapeDtypeStruct((NROWS128, 128), jnp.float32),
        compiler_params=pltpu.CompilerParams(
            dimension_semantics=("arbitrary",)),
    )(hidden)

    anchors, meta = _sc_select(norms2.reshape(N), tokens_i32, hidden)
    return anchors, meta


def kernel(hidden, tokens):
    anchors, meta = _run(hidden, tokens.astype(jnp.int32))
    sel_tokens = meta[:K].astype(tokens.dtype)
    return anchors, sel_tokens


# speedup vs baseline: 1.4756x; 1.4756x over previous
import jax
import jax.numpy as jnp
from jax import lax
from jax.experimental import pallas as pl
from jax.experimental.pallas import tpu as pltpu

N = 8192
DIM = 4096
K = 3

BLK = 512
NSTREAM = 1
GRID_F = N // (BLK * NSTREAM)
SUB = BLK // 128
NROWS128 = N // 128


def _fused_body(*refs):
    h_blks = refs[:NSTREAM]
    tokens_ref, hid_any, anchors_ref, meta_ref, norms_ref, sem = refs[NSTREAM:]
    i = pl.program_id(0)
    for j, h_blk in enumerate(h_blks):
        x = h_blk[...]
        s = jnp.sum(x * x, axis=1)
        norms_ref[pl.ds((j * GRID_F + i) * SUB, SUB), :] = s.reshape(SUB, 128)

    @pl.when(i == GRID_F - 1)
    def _():
        v = norms_ref[...]
        row = lax.broadcasted_iota(jnp.int32, v.shape, 0)
        lane = lax.broadcasted_iota(jnp.int32, v.shape, 1)
        gidx = row * 128 + lane
        big = jnp.int32(2**31 - 1)

        idxs = []
        for _ in range(K):
            m = jnp.max(v)
            cand = jnp.where(v == m, gidx, big)
            ik = jnp.min(cand)
            idxs.append(ik)
            v = jnp.where(gidx == ik, jnp.float32(-1.0), v)

        t = tokens_ref[...]
        toks = [jnp.sum(jnp.where(gidx == ik, t, 0)) for ik in idxs]

        lane8 = lax.broadcasted_iota(jnp.int32, (8, 128), 1)
        meta_ref[...] = jnp.where(lane8 == 0, toks[0],
                                  jnp.where(lane8 == 1, toks[1],
                                            jnp.where(lane8 == 2, toks[2], 0)))

        cps = [pltpu.make_async_copy(hid_any.at[pl.ds(ik, 1), :],
                                     anchors_ref.at[pl.ds(k, 1), :], sem)
               for k, ik in enumerate(idxs)]
        for cp in cps:
            cp.start()
        for cp in cps:
            cp.wait()


def _make_in_spec(j):
    return pl.BlockSpec((BLK, DIM), lambda i, j=j: (j * GRID_F + i, 0))


@jax.jit
def _run(hidden, tokens_2d):
    anchors, meta = pl.pallas_call(
        _fused_body,
        grid=(GRID_F,),
        in_specs=[_make_in_spec(j) for j in range(NSTREAM)] + [
            pl.BlockSpec(memory_space=pltpu.VMEM),
            pl.BlockSpec(memory_space=pl.ANY),
        ],
        out_specs=[
            pl.BlockSpec((K, DIM), lambda i: (0, 0)),
            pl.BlockSpec((8, 128), lambda i: (0, 0)),
        ],
        out_shape=[
            jax.ShapeDtypeStruct((K, DIM), jnp.float32),
            jax.ShapeDtypeStruct((8, 128), jnp.int32),
        ],
        scratch_shapes=[
            pltpu.VMEM((NROWS128, 128), jnp.float32),
            pltpu.SemaphoreType.DMA,
        ],
        compiler_params=pltpu.CompilerParams(
            dimension_semantics=("arbitrary",)),
    )(*([hidden] * NSTREAM), tokens_2d, hidden)
    return anchors, meta


def kernel(hidden, tokens):
    tokens_2d = tokens.astype(jnp.int32).reshape(NROWS128, 128)
    anchors, meta = _run(hidden, tokens_2d)
    sel_tokens = meta[0, :K].astype(tokens.dtype)
    return anchors, sel_tokens
